# baseline (speedup 1.0000x reference)
import functools

import jax
from jax.experimental import pallas as pl
from jax.experimental.pallas import tpu as pltpu


def _add_body(x_ref, row_ref, col_ref, o_ref, emb_ref, *, d):
    @pl.when(pl.program_id(0) == 0)
    def _():
        emb_ref[:, :d] = row_ref[...]
        emb_ref[:, d:] = col_ref[...]

    o_ref[...] = x_ref[...] + emb_ref[...][None, :, :]


def kernel(inputs, row_table, col_table):
    B, H, W, C = inputs.shape
    d = row_table.shape[1]
    K = 32
    x = inputs.reshape(B * H, W, C)
    grid = (B * H // K,)
    out = pl.pallas_call(
        functools.partial(_add_body, d=d),
        grid=grid,
        in_specs=[
            pl.BlockSpec((K, W, C), lambda i: (i, 0, 0)),
            pl.BlockSpec((W, d), lambda i: (0, 0)),
            pl.BlockSpec((H, d), lambda i: (0, 0)),
        ],
        out_specs=pl.BlockSpec((K, W, C), lambda i: (i, 0, 0)),
        out_shape=jax.ShapeDtypeStruct((B * H, W, C), inputs.dtype),
        scratch_shapes=[pltpu.VMEM((W, C), inputs.dtype)],
    )(x, row_table, col_table)
    return out.reshape(B, H, W, C)
